# baseline (speedup 1.0000x reference)
import functools

import jax
import jax.numpy as jnp
from jax import lax
from jax.experimental import pallas as pl
from jax.experimental.pallas import tpu as pltpu
from jax.experimental.pallas import tpu_sc as plsc

NC = 2
NS = 16
LANES = 16
CHUNK = 128
NB = 4
HC = 64

_f32 = jnp.float32
_i32 = jnp.int32


def _mesh():
    return plsc.VectorSubcoreMesh(core_axis_name="c", subcore_axis_name="s",
                                  num_cores=NC, num_subcores=NS)


def _sc_segment_sum(table2, src_ids, dst_ids, zeros_hbm, acc_rows):
    e_pad = src_ids.shape[0]
    per_tile = e_pad // NS
    groups = per_tile // (CHUNK * NB)
    zrows = acc_rows // NS

    @functools.partial(
        pl.kernel,
        out_type=jax.ShapeDtypeStruct((NC, acc_rows, HC), _f32),
        mesh=_mesh(),
        scratch_types=[
            pltpu.VMEM_SHARED((acc_rows, HC), _f32),
            pltpu.VMEM((zrows, HC), _f32),
            [pltpu.VMEM((CHUNK,), _i32) for _ in range(NB)],
            [pltpu.VMEM((CHUNK,), _i32) for _ in range(NB)],
            [pltpu.VMEM((CHUNK, HC), _f32) for _ in range(NB)],
            pltpu.SemaphoreType.DMA,
            pltpu.SemaphoreType.DMA,
            pltpu.SemaphoreType.DMA,
        ],
        compiler_params=pltpu.CompilerParams(use_tc_tiling_on_sc=False),
    )
    def body(table_h, src_h, dst_h, zeros_h, out_h, acc, zbuf, sidx, didx,
             rows, sem_i, sem_g, sem_s):
        c = lax.axis_index("c")
        s = lax.axis_index("s")
        pltpu.sync_copy(zeros_h, zbuf)
        pltpu.sync_copy(zbuf, acc.at[pl.ds(s * zrows, zrows)])
        plsc.subcore_barrier()
        base = s * per_tile

        def group_body(g, carry):
            off = base + g * (CHUNK * NB)
            ds = [pltpu.async_copy(src_h.at[pl.ds(off + b * CHUNK, CHUNK)],
                                   sidx[b], sem_i) for b in range(NB)]
            ds += [pltpu.async_copy(dst_h.at[pl.ds(off + b * CHUNK, CHUNK)],
                                    didx[b], sem_i) for b in range(NB)]
            for d in ds:
                d.wait()
            ds = [pltpu.async_copy(table_h.at[c].at[sidx[b]], rows[b], sem_g)
                  for b in range(NB)]
            for d in ds:
                d.wait()
            ds = [pltpu.async_copy(rows[b], acc.at[didx[b]], sem_s, add=True)
                  for b in range(NB)]
            for d in ds:
                d.wait()
            return carry

        lax.fori_loop(0, groups, group_body, 0)
        plsc.subcore_barrier()
        pltpu.sync_copy(acc.at[pl.ds(s * zrows, zrows)], zbuf)
        pltpu.sync_copy(zbuf, out_h.at[c, pl.ds(s * zrows, zrows)])

    return body(table2, src_ids, dst_ids, zeros_hbm)


def _sc_segment_sum_rounds(table2, src_ids, dst_ids, zeros_hbm, out_rows, ch):
    e_pad = src_ids.shape[0]
    per_tile = e_pad // NS
    groups = per_tile // (CHUNK * NB)
    acc_rows = ch + CHUNK
    zrows = acc_rows // NS
    wrows = ch // NS
    rounds = out_rows // ch

    @functools.partial(
        pl.kernel,
        out_type=jax.ShapeDtypeStruct((NC, out_rows, HC), _f32),
        mesh=_mesh(),
        scratch_types=[
            pltpu.VMEM_SHARED((acc_rows, HC), _f32),
            pltpu.VMEM((zrows, HC), _f32),
            [pltpu.VMEM((CHUNK,), _i32) for _ in range(NB)],
            [pltpu.VMEM((CHUNK,), _i32) for _ in range(NB)],
            [pltpu.VMEM((CHUNK,), _i32) for _ in range(NB)],
            [pltpu.VMEM((CHUNK, HC), _f32) for _ in range(NB)],
            pltpu.SemaphoreType.DMA,
            pltpu.SemaphoreType.DMA,
            pltpu.SemaphoreType.DMA,
        ],
        compiler_params=pltpu.CompilerParams(use_tc_tiling_on_sc=False),
    )
    def body(table_h, src_h, dst_h, zeros_h, out_h, acc, zbuf, sidx, didx,
             lidx, rows, sem_i, sem_g, sem_s):
        c = lax.axis_index("c")
        s = lax.axis_index("s")
        base = s * per_tile

        def round_body(r, carry):
            lo = r * ch
            pltpu.sync_copy(zeros_h, zbuf)
            pltpu.sync_copy(zbuf, acc.at[pl.ds(s * zrows, zrows)])
            plsc.subcore_barrier()

            def group_body(g, icarry):
                off = base + g * (CHUNK * NB)
                ds = [pltpu.async_copy(
                    src_h.at[pl.ds(off + b * CHUNK, CHUNK)], sidx[b], sem_i)
                    for b in range(NB)]
                ds += [pltpu.async_copy(
                    dst_h.at[pl.ds(off + b * CHUNK, CHUNK)], didx[b], sem_i)
                    for b in range(NB)]
                for d in ds:
                    d.wait()
                ds = [pltpu.async_copy(table_h.at[c].at[sidx[b]], rows[b],
                                       sem_g) for b in range(NB)]
                for b in range(NB):
                    for j in range(CHUNK // LANES):
                        d = didx[b][pl.ds(j * LANES, LANES)]
                        ld = d - lo
                        ok = (ld >= 0) & (ld < ch)
                        lidx[b][pl.ds(j * LANES, LANES)] = jnp.where(ok, ld,
                                                                     ch)
                for d in ds:
                    d.wait()
                ds = [pltpu.async_copy(rows[b], acc.at[lidx[b]], sem_s,
                                       add=True) for b in range(NB)]
                for d in ds:
                    d.wait()
                return icarry

            lax.fori_loop(0, groups, group_body, 0)
            plsc.subcore_barrier()
            pltpu.sync_copy(acc.at[pl.ds(s * wrows, wrows)],
                            zbuf.at[pl.ds(0, wrows)])
            pltpu.sync_copy(zbuf.at[pl.ds(0, wrows)],
                            out_h.at[c, pl.ds(lo + s * wrows, wrows)])
            plsc.subcore_barrier()
            return carry

        lax.fori_loop(0, rounds, round_body, 0)

    return body(table2, src_ids, dst_ids, zeros_hbm)


_BM = 512


def _tc_call(fn, grid_m, in_shapes, out_shapes, args):
    in_specs = []
    for shp, blocked in in_shapes:
        if blocked:
            in_specs.append(pl.BlockSpec((_BM, shp[1]), lambda i: (i, 0)))
        else:
            in_specs.append(pl.BlockSpec(shp, lambda i: (0, 0)))
    out_specs = [pl.BlockSpec((_BM, shp[1]), lambda i: (i, 0))
                 for shp in out_shapes]
    out_shape = [jax.ShapeDtypeStruct(shp, _f32) for shp in out_shapes]
    if len(out_shapes) == 1:
        out_specs, out_shape = out_specs[0], out_shape[0]
    return pl.pallas_call(
        fn, grid=(grid_m,), in_specs=in_specs, out_specs=out_specs,
        out_shape=out_shape)(*args)


def _emb_body(x_ref, w_ref, o_ref):
    o_ref[...] = jnp.dot(x_ref[...], w_ref[...],
                         preferred_element_type=_f32)


def _enc_body(xh_ref, wd_ref, a_ref, wp_ref, xl_ref, msg_ref):
    dst = jnp.dot(xh_ref[...], wd_ref[...], preferred_element_type=_f32)
    xl = jax.nn.relu(dst + a_ref[...])
    xl_ref[...] = xl
    msg_ref[...] = jnp.dot(xl, wp_ref[...], preferred_element_type=_f32)


def _proc_body(xl_ref, p_ref, wd_ref, o_ref):
    xlp = 2.0 * xl_ref[...] + jax.nn.relu(p_ref[...])
    o_ref[...] = jnp.dot(xlp, wd_ref[...], preferred_element_type=_f32)


def _dec_body(agg_ref, semb_ref, wo_ref, xl_ref, o_ref):
    xd = jax.nn.relu(agg_ref[...] + jax.nn.relu(semb_ref[...]))
    o_ref[...] = jnp.dot(xd, wo_ref[...],
                         preferred_element_type=_f32) + xl_ref[...]


def _round_up(n, m):
    return ((n + m - 1) // m) * m


def _halves(a):
    n = a.shape[0]
    return jnp.transpose(a.reshape(n, NC, HC), (1, 0, 2))


def kernel(x, latlons_data, latlons_hidden, trainable_data, trainable_hidden,
           W_src_enc, W_dst_enc, W_proc, W_dec, W_out,
           edge_index_enc, edge_index_proc, edge_index_dec):
    b, t, e, nd, v = x.shape
    nh = latlons_hidden.shape[0]
    cdim = W_src_enc.shape[1]

    md = _round_up(nd, _BM)
    mh = _round_up(nh, _BM)

    x_flat = jnp.transpose(x, (0, 2, 3, 1, 4)).reshape(nd, t * v)
    attr_d = jnp.concatenate(
        [jnp.sin(latlons_data), jnp.cos(latlons_data), trainable_data], -1)
    x_data_latent = jnp.concatenate([x_flat, attr_d], -1)
    x_data_latent = jnp.pad(x_data_latent, ((0, md - nd), (0, 0)))
    xh_lat = jnp.concatenate(
        [jnp.sin(latlons_hidden), jnp.cos(latlons_hidden), trainable_hidden],
        -1)
    xh_lat = jnp.pad(xh_lat, ((0, mh - nh), (0, 0)))

    acc_h = _round_up(nh + 1, NS * 8)

    def pad_edges(ei, trash):
        ecnt = ei.shape[1]
        epad = _round_up(ecnt, NS * CHUNK * NB)
        src = jnp.concatenate([ei[0], jnp.zeros((epad - ecnt,), _i32)])
        dst = jnp.concatenate([ei[1], jnp.full((epad - ecnt,), trash, _i32)])
        return src, dst

    enc_src, enc_dst = pad_edges(edge_index_enc, nh)
    proc_src, proc_dst = pad_edges(edge_index_proc, nh)
    dec_src, dec_dst = pad_edges(edge_index_dec, nd)

    zeros_h = jnp.zeros((acc_h // NS, HC), _f32)

    ch = 10112
    dec_rows = _round_up(md, ch)
    zeros_d = jnp.zeros(((ch + CHUNK) // NS, HC), _f32)

    src_emb = _tc_call(_emb_body, md // _BM,
                       [((md, x_data_latent.shape[1]), True),
                        (tuple(W_src_enc.shape), False)],
                       [(md, cdim)], [x_data_latent, W_src_enc])

    agg_enc = _sc_segment_sum(_halves(src_emb), enc_src, enc_dst, zeros_h,
                              acc_h)
    a = jnp.transpose(agg_enc, (1, 0, 2)).reshape(acc_h, cdim)
    a = jnp.pad(a[:nh], ((0, mh - nh), (0, 0)))

    xl, msg = _tc_call(_enc_body, mh // _BM,
                       [((mh, xh_lat.shape[1]), True),
                        (tuple(W_dst_enc.shape), False),
                        ((mh, cdim), True),
                        (tuple(W_proc.shape), False)],
                       [(mh, cdim), (mh, cdim)],
                       [xh_lat, W_dst_enc, a, W_proc])

    agg_p = _sc_segment_sum(_halves(msg), proc_src, proc_dst, zeros_h, acc_h)
    p = jnp.transpose(agg_p, (1, 0, 2)).reshape(acc_h, cdim)
    p = jnp.pad(p[:nh], ((0, mh - nh), (0, 0)))

    dmsg = _tc_call(_proc_body, mh // _BM,
                    [((mh, cdim), True), ((mh, cdim), True),
                     (tuple(W_dec.shape), False)],
                    [(mh, cdim)], [xl, p, W_dec])

    agg_d = _sc_segment_sum_rounds(_halves(dmsg), dec_src, dec_dst, zeros_d,
                                   dec_rows, ch)
    agg_d = jnp.transpose(agg_d, (1, 0, 2)).reshape(dec_rows, cdim)[:md]

    vpad = _round_up(v, 64)
    w_out_p = jnp.pad(W_out, ((0, 0), (0, vpad - v)))
    x_last = jnp.pad(x[0, -1, 0], ((0, md - nd), (0, vpad - v)))
    out = _tc_call(_dec_body, md // _BM,
                   [((md, cdim), True), ((md, cdim), True),
                    ((cdim, vpad), False), ((md, vpad), True)],
                   [(md, vpad)], [agg_d, src_emb, w_out_p, x_last])

    return out[:nd, :v].reshape(b, e, nd, v)


# speedup vs baseline: 1.8975x; 1.8975x over previous
import functools

import jax
import jax.numpy as jnp
from jax import lax
from jax.experimental import pallas as pl
from jax.experimental.pallas import tpu as pltpu
from jax.experimental.pallas import tpu_sc as plsc

NC = 2
NS = 16
LANES = 16
CHUNK = 128
NB = 4
NBD = 2
HC = 64

_f32 = jnp.float32
_i32 = jnp.int32


def _mesh():
    return plsc.VectorSubcoreMesh(core_axis_name="c", subcore_axis_name="s",
                                  num_cores=NC, num_subcores=NS)


def _sc_segment_sum(table2, src_ids, dst_ids, zeros_hbm, acc_rows):
    e_pad = src_ids.shape[0]
    per_tile = e_pad // NS
    groups = per_tile // (CHUNK * NB)
    zrows = acc_rows // NS

    @functools.partial(
        pl.kernel,
        out_type=jax.ShapeDtypeStruct((NC, acc_rows, HC), _f32),
        mesh=_mesh(),
        scratch_types=[
            pltpu.VMEM_SHARED((acc_rows, HC), _f32),
            pltpu.VMEM((zrows, HC), _f32),
            [pltpu.VMEM((CHUNK,), _i32) for _ in range(NB)],
            [pltpu.VMEM((CHUNK,), _i32) for _ in range(NB)],
            [pltpu.VMEM((CHUNK, HC), _f32) for _ in range(NB)],
            pltpu.SemaphoreType.DMA,
            pltpu.SemaphoreType.DMA,
            pltpu.SemaphoreType.DMA,
        ],
        compiler_params=pltpu.CompilerParams(use_tc_tiling_on_sc=False),
    )
    def body(table_h, src_h, dst_h, zeros_h, out_h, acc, zbuf, sidx, didx,
             rows, sem_i, sem_g, sem_s):
        c = lax.axis_index("c")
        s = lax.axis_index("s")
        pltpu.sync_copy(zeros_h, zbuf)
        pltpu.sync_copy(zbuf, acc.at[pl.ds(s * zrows, zrows)])
        plsc.subcore_barrier()
        base = s * per_tile

        def group_body(g, carry):
            off = base + g * (CHUNK * NB)
            ds = [pltpu.async_copy(src_h.at[pl.ds(off + b * CHUNK, CHUNK)],
                                   sidx[b], sem_i) for b in range(NB)]
            ds += [pltpu.async_copy(dst_h.at[pl.ds(off + b * CHUNK, CHUNK)],
                                    didx[b], sem_i) for b in range(NB)]
            for d in ds:
                d.wait()
            ds = [pltpu.async_copy(table_h.at[c].at[sidx[b]], rows[b], sem_g)
                  for b in range(NB)]
            for d in ds:
                d.wait()
            ds = [pltpu.async_copy(rows[b], acc.at[didx[b]], sem_s, add=True)
                  for b in range(NB)]
            for d in ds:
                d.wait()
            return carry

        lax.fori_loop(0, groups, group_body, 0)
        plsc.subcore_barrier()
        pltpu.sync_copy(acc.at[pl.ds(s * zrows, zrows)], zbuf)
        pltpu.sync_copy(zbuf, out_h.at[c, pl.ds(s * zrows, zrows)])

    return body(table2, src_ids, dst_ids, zeros_hbm)


def _sc_segment_sum_rounds(table2, src_ids, dst_ids, zeros_hbm, out_rows, ch):
    e_pad = src_ids.shape[0]
    per_tile = e_pad // NS
    gsz = CHUNK * NBD
    groups = per_tile // gsz
    acc_rows = ch + CHUNK
    zrows = acc_rows // NS
    wrows = ch // NS
    rounds = out_rows // ch
    clen = per_tile + gsz
    wfull = wrows // CHUNK
    wtail = wrows - wfull * CHUNK

    @functools.partial(
        pl.kernel,
        out_type=jax.ShapeDtypeStruct((NC, out_rows, HC), _f32),
        mesh=_mesh(),
        scratch_types=[
            pltpu.VMEM_SHARED((acc_rows, HC), _f32),
            pltpu.VMEM((CHUNK, HC), _f32),
            [pltpu.VMEM((CHUNK,), _i32) for _ in range(NBD)],
            [pltpu.VMEM((CHUNK,), _i32) for _ in range(NBD)],
            pltpu.VMEM((clen,), _i32),
            [pltpu.VMEM((CHUNK, HC), _f32) for _ in range(NBD)],
            pltpu.SemaphoreType.DMA,
            pltpu.SemaphoreType.DMA,
            pltpu.SemaphoreType.DMA,
        ],
        compiler_params=pltpu.CompilerParams(use_tc_tiling_on_sc=False,
                                             needs_layout_passes=False),
    )
    def body(table_h, src_h, dst_h, zeros_h, out_h, acc, zbuf, sidx, didx,
             cpack, rows, sem_i, sem_g, sem_s):
        c = lax.axis_index("c")
        s = lax.axis_index("s")
        base = s * per_tile

        def round_body(r, carry):
            lo = r * ch
            pltpu.sync_copy(zeros_h, zbuf)
            for k in range(zrows // CHUNK):
                pltpu.sync_copy(zbuf, acc.at[pl.ds(s * zrows + k * CHUNK,
                                                   CHUNK)])
            plsc.subcore_barrier()

            def scan_body(g, cnt):
                off = base + g * gsz
                ds = [pltpu.async_copy(
                    src_h.at[pl.ds(off + b * CHUNK, CHUNK)], sidx[b], sem_i)
                    for b in range(NBD)]
                ds += [pltpu.async_copy(
                    dst_h.at[pl.ds(off + b * CHUNK, CHUNK)], didx[b], sem_i)
                    for b in range(NBD)]
                for d in ds:
                    d.wait()
                for b in range(NBD):
                    for j in range(CHUNK // LANES):
                        dv = didx[b][pl.ds(j * LANES, LANES)]
                        sv = sidx[b][pl.ds(j * LANES, LANES)]
                        ld = dv - lo
                        ok = (ld >= 0) & (ld < ch)
                        pk = (ld << 14) | sv
                        plsc.store_compressed(cpack.at[pl.ds(cnt, LANES)],
                                              pk, mask=ok)
                        cnt = cnt + jnp.sum(ok.astype(_i32))
                return cnt

            cnt = lax.fori_loop(0, groups, scan_body, jnp.int32(0))

            ngroups_r = (cnt + (gsz - 1)) // gsz
            trash_pk = ch << 14

            @pl.when(cnt > 0)
            def _():
                tbase = (ngroups_r - 1) * gsz
                lane = lax.iota(_i32, LANES)
                for k in range(gsz // LANES):
                    off = tbase + k * LANES
                    sel = (off + lane) < cnt
                    pv = cpack[pl.ds(off, LANES)]
                    cpack[pl.ds(off, LANES)] = jnp.where(sel, pv, trash_pk)

            def gs_body(g, icarry):
                off = g * gsz
                for b in range(NBD):
                    for j in range(CHUNK // LANES):
                        pv = cpack[pl.ds(off + b * CHUNK + j * LANES, LANES)]
                        sidx[b][pl.ds(j * LANES, LANES)] = pv & 16383
                        didx[b][pl.ds(j * LANES, LANES)] = pv >> 14
                ds = [pltpu.async_copy(table_h.at[c].at[sidx[b]], rows[b],
                                       sem_g) for b in range(NBD)]
                for d in ds:
                    d.wait()
                ds = [pltpu.async_copy(rows[b], acc.at[didx[b]], sem_s,
                                       add=True) for b in range(NBD)]
                for d in ds:
                    d.wait()
                return icarry

            lax.fori_loop(0, ngroups_r, gs_body, 0)
            plsc.subcore_barrier()

            wbase = s * wrows
            obase = lo + s * wrows
            strips = [CHUNK] * wfull + ([wtail] if wtail else [])
            for i0 in range(0, len(strips), NBD):
                wave = strips[i0:i0 + NBD]
                bufs = [rows[k] if sz == CHUNK else zbuf
                        for k, sz in enumerate(wave)]
                ds = [pltpu.async_copy(
                    acc.at[pl.ds(wbase + (i0 + k) * CHUNK, sz)],
                    bufs[k].at[pl.ds(0, sz)], sem_g)
                    for k, sz in enumerate(wave)]
                for d in ds:
                    d.wait()
                ds = [pltpu.async_copy(
                    bufs[k].at[pl.ds(0, sz)],
                    out_h.at[c, pl.ds(obase + (i0 + k) * CHUNK, sz)], sem_s)
                    for k, sz in enumerate(wave)]
                for d in ds:
                    d.wait()
            plsc.subcore_barrier()
            return carry

        lax.fori_loop(0, rounds, round_body, 0)

    return body(table2, src_ids, dst_ids, zeros_hbm)


_BM = 512


def _tc_call(fn, grid_m, in_shapes, out_shapes, args):
    in_specs = []
    for shp, blocked in in_shapes:
        if blocked:
            in_specs.append(pl.BlockSpec((_BM, shp[1]), lambda i: (i, 0)))
        else:
            in_specs.append(pl.BlockSpec(shp, lambda i: (0, 0)))
    out_specs = [pl.BlockSpec((_BM, shp[1]), lambda i: (i, 0))
                 for shp in out_shapes]
    out_shape = [jax.ShapeDtypeStruct(shp, _f32) for shp in out_shapes]
    if len(out_shapes) == 1:
        out_specs, out_shape = out_specs[0], out_shape[0]
    return pl.pallas_call(
        fn, grid=(grid_m,), in_specs=in_specs, out_specs=out_specs,
        out_shape=out_shape)(*args)


def _emb_body(x_ref, w_ref, o_ref):
    o_ref[...] = jnp.dot(x_ref[...], w_ref[...],
                         preferred_element_type=_f32)


def _enc_body(xh_ref, wd_ref, a_ref, wp_ref, xl_ref, msg_ref):
    dst = jnp.dot(xh_ref[...], wd_ref[...], preferred_element_type=_f32)
    xl = jax.nn.relu(dst + a_ref[...])
    xl_ref[...] = xl
    msg_ref[...] = jnp.dot(xl, wp_ref[...], preferred_element_type=_f32)


def _proc_body(xl_ref, p_ref, wd_ref, o_ref):
    xlp = 2.0 * xl_ref[...] + jax.nn.relu(p_ref[...])
    o_ref[...] = jnp.dot(xlp, wd_ref[...], preferred_element_type=_f32)


def _dec_body(agg_ref, semb_ref, wo_ref, xl_ref, o_ref):
    xd = jax.nn.relu(agg_ref[...] + jax.nn.relu(semb_ref[...]))
    o_ref[...] = jnp.dot(xd, wo_ref[...],
                         preferred_element_type=_f32) + xl_ref[...]


def _round_up(n, m):
    return ((n + m - 1) // m) * m


def _halves(a):
    n = a.shape[0]
    return jnp.transpose(a.reshape(n, NC, HC), (1, 0, 2))


def kernel(x, latlons_data, latlons_hidden, trainable_data, trainable_hidden,
           W_src_enc, W_dst_enc, W_proc, W_dec, W_out,
           edge_index_enc, edge_index_proc, edge_index_dec):
    b, t, e, nd, v = x.shape
    nh = latlons_hidden.shape[0]
    cdim = W_src_enc.shape[1]

    md = _round_up(nd, _BM)
    mh = _round_up(nh, _BM)

    x_flat = jnp.transpose(x, (0, 2, 3, 1, 4)).reshape(nd, t * v)
    attr_d = jnp.concatenate(
        [jnp.sin(latlons_data), jnp.cos(latlons_data), trainable_data], -1)
    x_data_latent = jnp.concatenate([x_flat, attr_d], -1)
    x_data_latent = jnp.pad(x_data_latent, ((0, md - nd), (0, 0)))
    xh_lat = jnp.concatenate(
        [jnp.sin(latlons_hidden), jnp.cos(latlons_hidden), trainable_hidden],
        -1)
    xh_lat = jnp.pad(xh_lat, ((0, mh - nh), (0, 0)))

    acc_h = _round_up(nh + 1, NS * 8)

    def pad_edges(ei, trash, nb):
        ecnt = ei.shape[1]
        epad = _round_up(ecnt, NS * CHUNK * nb)
        src = jnp.concatenate([ei[0], jnp.zeros((epad - ecnt,), _i32)])
        dst = jnp.concatenate([ei[1], jnp.full((epad - ecnt,), trash, _i32)])
        return src, dst

    enc_src, enc_dst = pad_edges(edge_index_enc, nh, NB)
    proc_src, proc_dst = pad_edges(edge_index_proc, nh, NB)
    dec_src, dec_dst = pad_edges(edge_index_dec, nd, NBD)

    zeros_h = jnp.zeros((acc_h // NS, HC), _f32)

    ch = 10112
    dec_rows = _round_up(md, ch)
    zeros_d = jnp.zeros((CHUNK, HC), _f32)

    src_emb = _tc_call(_emb_body, md // _BM,
                       [((md, x_data_latent.shape[1]), True),
                        (tuple(W_src_enc.shape), False)],
                       [(md, cdim)], [x_data_latent, W_src_enc])

    agg_enc = _sc_segment_sum(_halves(src_emb), enc_src, enc_dst, zeros_h,
                              acc_h)
    a = jnp.transpose(agg_enc, (1, 0, 2)).reshape(acc_h, cdim)
    a = jnp.pad(a[:nh], ((0, mh - nh), (0, 0)))

    xl, msg = _tc_call(_enc_body, mh // _BM,
                       [((mh, xh_lat.shape[1]), True),
                        (tuple(W_dst_enc.shape), False),
                        ((mh, cdim), True),
                        (tuple(W_proc.shape), False)],
                       [(mh, cdim), (mh, cdim)],
                       [xh_lat, W_dst_enc, a, W_proc])

    agg_p = _sc_segment_sum(_halves(msg), proc_src, proc_dst, zeros_h, acc_h)
    p = jnp.transpose(agg_p, (1, 0, 2)).reshape(acc_h, cdim)
    p = jnp.pad(p[:nh], ((0, mh - nh), (0, 0)))

    dmsg = _tc_call(_proc_body, mh // _BM,
                    [((mh, cdim), True), ((mh, cdim), True),
                     (tuple(W_dec.shape), False)],
                    [(mh, cdim)], [xl, p, W_dec])

    agg_d = _sc_segment_sum_rounds(_halves(dmsg), dec_src, dec_dst, zeros_d,
                                   dec_rows, ch)
    agg_d = jnp.transpose(agg_d, (1, 0, 2)).reshape(dec_rows, cdim)[:md]

    vpad = _round_up(v, 64)
    w_out_p = jnp.pad(W_out, ((0, 0), (0, vpad - v)))
    x_last = jnp.pad(x[0, -1, 0], ((0, md - nd), (0, vpad - v)))
    out = _tc_call(_dec_body, md // _BM,
                   [((md, cdim), True), ((md, cdim), True),
                    ((cdim, vpad), False), ((md, vpad), True)],
                   [(md, vpad)], [agg_d, src_emb, w_out_p, x_last])

    return out[:nd, :v].reshape(b, e, nd, v)
